# Initial kernel scaffold; baseline (speedup 1.0000x reference)
import jax
import jax.numpy as jnp
from jax.experimental import pallas as pl


def kernel(input_index, embeds, graph):
    raise NotImplementedError("write your pallas kernel here")


# speedup vs baseline: 24.5979x; 24.5979x over previous
import functools

import jax
import jax.numpy as jnp
from jax import lax
from jax.experimental import pallas as pl
from jax.experimental.pallas import tpu as pltpu
from jax.experimental.pallas import tpu_sc as plsc

_NUM_POINTS = 4096
_DIMS = 128
_BATCH = 512
_NC = 2
_NS = 16
_NW = _NC * _NS
_RPW = _BATCH // _NW
_L = 16


def _sc_gather_body(idx_hbm, embeds_hbm, graph_hbm, eb_out, g_out,
                    idx_v, myidx_v, erows_v, grows_v, gstage_v, sem):
    wid = lax.axis_index("s") * _NC + lax.axis_index("c")
    base = wid * _RPW
    pltpu.sync_copy(idx_hbm, idx_v)
    pltpu.sync_copy(idx_hbm.at[pl.ds(base, _RPW)], myidx_v)
    cp_e = pltpu.async_copy(embeds_hbm.at[myidx_v], erows_v, sem)
    cp_g = pltpu.async_copy(graph_hbm.at[myidx_v], grows_v, sem)
    cp_e.wait()
    cp_g.wait()
    pltpu.sync_copy(erows_v, eb_out.at[pl.ds(base, _RPW)])

    @pl.loop(0, _RPW)
    def _row(r):
        row_sel = jnp.full((_L,), r, dtype=jnp.int32)
        for c in range(_BATCH // _L):
            cols = idx_v[pl.ds(c * _L, _L)]
            vals = plsc.load_gather(grows_v, [row_sel, cols])
            gstage_v[r, pl.ds(c * _L, _L)] = vals

    pltpu.sync_copy(gstage_v, g_out.at[pl.ds(base, _RPW)])


@functools.cache
def _build_sc_gather():
    mesh = plsc.VectorSubcoreMesh(core_axis_name="c", subcore_axis_name="s")
    return pl.kernel(
        _sc_gather_body,
        out_type=[
            jax.ShapeDtypeStruct((_BATCH, _DIMS), jnp.float32),
            jax.ShapeDtypeStruct((_BATCH, _BATCH), jnp.float32),
        ],
        mesh=mesh,
        compiler_params=pltpu.CompilerParams(use_tc_tiling_on_sc=False,
                                             needs_layout_passes=False),
        scratch_types=[
            pltpu.VMEM((_BATCH,), jnp.int32),
            pltpu.VMEM((_RPW,), jnp.int32),
            pltpu.VMEM((_RPW, _DIMS), jnp.float32),
            pltpu.VMEM((_RPW, _NUM_POINTS), jnp.float32),
            pltpu.VMEM((_RPW, _BATCH), jnp.float32),
            pltpu.SemaphoreType.DMA,
        ],
    )


def _tc_loss_body(eb_ref, g_ref, out_ref):
    eb = eb_ref[...]
    g = g_ref[...]
    gram = lax.dot_general(eb, eb, (((1,), (1,)), ((), ())),
                           preferred_element_type=jnp.float32,
                           precision=lax.Precision.HIGHEST)
    row = lax.broadcasted_iota(jnp.int32, (_BATCH, _BATCH), 0)
    col = lax.broadcasted_iota(jnp.int32, (_BATCH, _BATCH), 1)
    diag = jnp.where(row == col, gram, 0.0)
    n_row = jnp.sum(diag, axis=1, keepdims=True)
    n_col = jnp.sum(diag, axis=0, keepdims=True)
    d2 = jnp.maximum(n_row + n_col - 2.0 * gram, 0.0) + 1e-12
    loss = jnp.abs(d2 / (g * g) - 1.0)
    loss = jnp.where(col > row, loss, 0.0)
    out_ref[0, 0] = jnp.sum(loss)


def _tc_loss(eb, g):
    return pl.pallas_call(
        _tc_loss_body,
        out_shape=jax.ShapeDtypeStruct((1, 1), jnp.float32),
        out_specs=pl.BlockSpec(memory_space=pltpu.SMEM),
    )(eb, g)


def kernel(input_index, embeds, graph):
    idx = input_index.astype(jnp.int32)
    eb, g = _build_sc_gather()(idx, embeds, graph)
    return _tc_loss(eb, g)[0, 0]


# speedup vs baseline: 61.8174x; 2.5131x over previous
import functools

import jax
import jax.numpy as jnp
from jax import lax
from jax.experimental import pallas as pl
from jax.experimental.pallas import tpu as pltpu
from jax.experimental.pallas import tpu_sc as plsc

_NUM_POINTS = 4096
_DIMS = 128
_BATCH = 512
_NC = 2
_NS = 16
_NW = _NC * _NS
_RPW = _BATCH // _NW
_L = 16
_CPR = _NUM_POINTS // _DIMS
_CPB = _BATCH // _DIMS


def _sc_select_body(idx_hbm, embeds_hbm, gt_hbm, eb_out, g_out,
                    idx_v, myidx_v, erows_v, idxbuf_v, hi_v, lo_v,
                    grows_v, gstage_v, sem):
    wid = lax.axis_index("s") * _NC + lax.axis_index("c")
    base = wid * _RPW
    pltpu.sync_copy(idx_hbm, idx_v)
    pltpu.sync_copy(idx_hbm.at[pl.ds(base, _RPW)], myidx_v)
    cp_e = pltpu.async_copy(embeds_hbm.at[myidx_v], erows_v, sem)

    trowbase = (myidx_v[...] >> 3) * (_CPR * 8) + (myidx_v[...] & 7)

    @pl.loop(0, _CPR, unroll=8)
    def _bld(c):
        idxbuf_v[c >> 3, pl.ds((c & 7) * _L, _L)] = trowbase + c * 8

    cps = [pltpu.async_copy(gt_hbm.at[idxbuf_v.at[k]],
                            grows_v.at[pl.ds(k * 128, 128)], sem)
           for k in range(4)]

    @pl.loop(0, _BATCH // _L, unroll=8)
    def _addr(c):
        cols = idx_v[pl.ds(c * _L, _L)]
        hi_v[pl.ds(c * _L, _L)] = (cols >> 7) * _RPW
        lo_v[pl.ds(c * _L, _L)] = cols & 127

    for cp in cps:
        cp.wait()
    cp_e.wait()
    pltpu.sync_copy(erows_v, eb_out.at[pl.ds(base, _RPW)])

    @pl.loop(0, _RPW)
    def _row(rl):
        rsel = jnp.full((_L,), rl, dtype=jnp.int32)
        krow = (rl >> 3) * (_CPB * 8) + (rl & 7)

        @pl.loop(0, _BATCH // _L, unroll=8)
        def _sel(jc):
            a = hi_v[pl.ds(jc * _L, _L)] + rsel
            vals = plsc.load_gather(grows_v, [a, lo_v[pl.ds(jc * _L, _L)]])
            gstage_v[krow + (jc >> 3) * 8, pl.ds((jc & 7) * _L, _L)] = vals

    pltpu.sync_copy(gstage_v, g_out.at[pl.ds(wid * (_CPB * _RPW), _CPB * _RPW)])


@functools.cache
def _build_sc_select():
    mesh = plsc.VectorSubcoreMesh(core_axis_name="c", subcore_axis_name="s")
    return pl.kernel(
        _sc_select_body,
        out_type=[
            jax.ShapeDtypeStruct((_BATCH, _DIMS), jnp.float32),
            jax.ShapeDtypeStruct((_BATCH * _CPB, _DIMS), jnp.float32),
        ],
        mesh=mesh,
        compiler_params=pltpu.CompilerParams(use_tc_tiling_on_sc=False,
                                             needs_layout_passes=False),
        scratch_types=[
            pltpu.VMEM((_BATCH,), jnp.int32),
            pltpu.VMEM((_RPW,), jnp.int32),
            pltpu.VMEM((_RPW, _DIMS), jnp.float32),
            pltpu.VMEM((4, 128), jnp.int32),
            pltpu.VMEM((_BATCH,), jnp.int32),
            pltpu.VMEM((_BATCH,), jnp.int32),
            pltpu.VMEM((_CPR * _RPW, _DIMS), jnp.float32),
            pltpu.VMEM((_CPB * _RPW, _DIMS), jnp.float32),
            pltpu.SemaphoreType.DMA,
        ],
    )


def _tc_loss_body(eb_ref, g_ref, out_ref):
    eb = eb_ref[...]
    g = g_ref[...]
    gram = lax.dot_general(eb, eb, (((1,), (1,)), ((), ())),
                           preferred_element_type=jnp.float32,
                           precision=lax.Precision.HIGHEST)
    row = lax.broadcasted_iota(jnp.int32, (_BATCH, _BATCH), 0)
    col = lax.broadcasted_iota(jnp.int32, (_BATCH, _BATCH), 1)
    diag = jnp.where(row == col, gram, 0.0)
    n_row = jnp.sum(diag, axis=1, keepdims=True)
    n_col = jnp.sum(diag, axis=0, keepdims=True)
    d2 = jnp.maximum(n_row + n_col - 2.0 * gram, 0.0) + 1e-12
    loss = jnp.abs(d2 / (g * g) - 1.0)
    loss = jnp.where(col > row, loss, 0.0)
    out_ref[0, 0] = jnp.sum(loss)


def _tc_loss(eb, g):
    return pl.pallas_call(
        _tc_loss_body,
        out_shape=jax.ShapeDtypeStruct((1, 1), jnp.float32),
        out_specs=pl.BlockSpec(memory_space=pltpu.SMEM),
    )(eb, g)


def kernel(input_index, embeds, graph):
    idx = input_index.astype(jnp.int32)
    gt = (graph.reshape(_NUM_POINTS // 8, 8, _CPR, _DIMS)
          .transpose(0, 2, 1, 3)
          .reshape(_NUM_POINTS // 8 * _CPR * 8, _DIMS))
    eb, gq = _build_sc_select()(idx, embeds, gt)
    g = (gq.reshape(_BATCH // 8, _CPB, 8, _DIMS)
         .transpose(0, 2, 1, 3)
         .reshape(_BATCH, _BATCH))
    return _tc_loss(eb, g)[0, 0]
